# baseline (speedup 1.0000x reference)
import functools

import jax
import jax.numpy as jnp
from jax import lax
from jax.experimental import pallas as pl
from jax.experimental.pallas import tpu as pltpu
from jax.experimental.pallas import tpu_sc as plsc

_N_EMBED = 128
_BATCH = 16384
_N_SAMPLES = 3
_NOISE = _BATCH * _N_SAMPLES

_NC, _NS = 2, 16
_NW = _NC * _NS
_CH = 128
_CB = _BATCH // (_NW * _CH)
_CZ = _NOISE // (_NW * _CH)
_ZPW = _CZ * _CH

_mesh = plsc.VectorSubcoreMesh(core_axis_name="c", subcore_axis_name="s")


@functools.partial(
    pl.kernel,
    mesh=_mesh,
    out_type=(
        jax.ShapeDtypeStruct((_BATCH // _CH, _CH, _N_EMBED), jnp.float32),
        jax.ShapeDtypeStruct((_BATCH // _CH, _CH, _N_EMBED), jnp.float32),
        jax.ShapeDtypeStruct((_NOISE // _CH, _CH, _N_EMBED), jnp.float32),
    ),
    scratch_types=[
        pltpu.VMEM((_CB, _CH), jnp.int32),
        pltpu.VMEM((_CB, _CH), jnp.int32),
        pltpu.VMEM((_ZPW,), jnp.int32),
        pltpu.VMEM((_ZPW,), jnp.int32),
        pltpu.VMEM((3, 2, _CH, _N_EMBED), jnp.float32),
        [pltpu.SemaphoreType.DMA] * 6,
        [pltpu.SemaphoreType.DMA] * 3,
    ],
    compiler_params=pltpu.CompilerParams(needs_layout_passes=False),
)
def _sc_gather(in_tab, out_tab, iw, ow, zw, iv, ov, nv,
               bi, bo, bzr, bz, rows, gsem, ssem):
    wid = lax.axis_index("s") * _NC + lax.axis_index("c")
    pltpu.sync_copy(iw.at[wid], bi)
    pltpu.sync_copy(ow.at[wid], bo)
    pltpu.sync_copy(zw.at[pl.ds(wid * _ZPW, _ZPW)], bzr)
    items = []
    for tab, buf, nch, dst in (
        (in_tab, bi, _CB, iv),
        (out_tab, bo, _CB, ov),
    ):
        for c in range(0, nch, 2):
            items.append((tab, buf.at[c], buf.at[c + 1],
                          dst.at[pl.ds(wid * nch + c, 2)]))
    for t in range(_N_SAMPLES):
        for c in range(0, _CB, 2):
            items.append((out_tab,
                          bz.at[pl.ds((t * _CB + c) * _CH, _CH)],
                          bz.at[pl.ds((t * _CB + c + 1) * _CH, _CH)],
                          nv.at[pl.ds(t * (_BATCH // _CH) + wid * _CB + c, 2)]))
    n = len(items)
    g_cp = [None] * n
    s_cp = [None] * n

    def start_gather(j):
        tab, idxa, idxb, _ = items[j]
        g_cp[j] = (
            pltpu.async_copy(tab.at[idxa], rows.at[j % 3, 0], gsem[(j % 3) * 2]),
            pltpu.async_copy(tab.at[idxb], rows.at[j % 3, 1], gsem[(j % 3) * 2 + 1]),
        )

    def start_store(j):
        dst = items[j][3]
        s_cp[j] = pltpu.async_copy(rows.at[j % 3], dst, ssem[j % 3])

    def deinterleave():
        iota3 = lax.iota(jnp.int32, 16) * 3
        for t in range(_N_SAMPLES):
            for k in range(_ZPW // _N_SAMPLES // 16):
                q = iota3 + (48 * k + t)
                v = plsc.load_gather(bzr, [q])
                bz[pl.ds(t * (_ZPW // _N_SAMPLES) + 16 * k, 16)] = v

    _D = 2
    start_gather(0)
    deinterleave()
    start_gather(1)
    for j in range(n):
        if j + _D < n:
            if j + _D - 3 >= 0:
                s_cp[j + _D - 3].wait()
            start_gather(j + _D)
        g_cp[j][0].wait()
        g_cp[j][1].wait()
        start_store(j)
    for j in range(n - 3, n):
        s_cp[j].wait()


def kernel(input_words, output_words, noise_words, in_embed_weight, out_embed_weight):
    iw = input_words.astype(jnp.int32).reshape(_NW, _CB, _CH)
    ow = output_words.astype(jnp.int32).reshape(_NW, _CB, _CH)
    zw = noise_words.astype(jnp.int32)
    iv, ov, nv = _sc_gather(in_embed_weight, out_embed_weight, iw, ow, zw)
    return (iv.reshape(_BATCH, _N_EMBED),
            ov.reshape(_BATCH, _N_EMBED),
            nv.reshape(_N_SAMPLES, _BATCH, _N_EMBED).transpose(1, 0, 2))
